# baseline (speedup 1.0000x reference)
import jax
import jax.numpy as jnp
from jax.experimental import pallas as pl

_N = 1024
_F = 64
_H = 128
_BI = 8


def _node_kernel(nf, W1, b1, W2, b2, W4, b4, h_out, hT_out):
    h0 = nf[...]
    t = jnp.dot(h0, W1[...], preferred_element_type=jnp.float32) + b1[...]
    t = jnp.where(t >= 0.0, t, 0.01 * t)
    t = jnp.dot(t, W2[...], preferred_element_type=jnp.float32) + b2[...]
    t = jnp.where(t >= 0.0, t, 0.01 * t)
    t = jnp.dot(t, W4[...], preferred_element_type=jnp.float32) + b4[...]
    h = t + h0
    h_out[...] = h
    hT_out[...] = h.T.astype(jnp.bfloat16)


def _edge_kernel(hi, hT, W5Tt, b5c, w6c, b6d, out0, out1):
    g = pl.program_id(0)
    hrep = jax.lax.broadcast_in_dim(hi[...], (_BI, _H, _F), (0, 2))
    hrep = hrep.reshape(_BI * _H, _F)
    A = (hrep * W5Tt[...]).astype(jnp.bfloat16)
    T = jnp.dot(A, hT[...], preferred_element_type=jnp.float32)
    T = jnp.maximum(T + b5c[...], 0.0)
    U = T * w6c[...]
    D = jnp.sum(U.reshape(_BI, _H, _N), axis=1) + b6d[0, 0]
    p1 = jax.nn.sigmoid(D)
    p0 = 1.0 - p1
    row = jax.lax.broadcasted_iota(jnp.int32, (_BI, _N), 0)
    col = jax.lax.broadcasted_iota(jnp.int32, (_BI, _N), 1)
    diag = col == (g * _BI + row)
    out0[...] = jnp.where(diag, 0.0, p0)
    out1[...] = jnp.where(diag, 0.0, p1)


def kernel(x, node_features, W1, b1, W2, b2, W4, b4, W5, b5, W6, b6):
    f32 = jnp.float32
    h, hT = pl.pallas_call(
        _node_kernel,
        out_shape=(
            jax.ShapeDtypeStruct((_N, _F), f32),
            jax.ShapeDtypeStruct((_F, _N), jnp.bfloat16),
        ),
    )(
        node_features,
        W1,
        b1.reshape(1, _H),
        W2,
        b2.reshape(1, _H),
        W4,
        b4.reshape(1, _F),
    )

    W5Tt = jnp.tile(W5.T, (_BI, 1))
    b5c = jnp.tile(b5, _BI).reshape(_BI * _H, 1)
    w6c = jnp.tile(W6[:, 1] - W6[:, 0], _BI).reshape(_BI * _H, 1)
    b6d = (b6[1] - b6[0]).reshape(1, 1)

    out0, out1 = pl.pallas_call(
        _edge_kernel,
        grid=(_N // _BI,),
        in_specs=[
            pl.BlockSpec((_BI, _F), lambda g: (g, 0)),
            pl.BlockSpec((_F, _N), lambda g: (0, 0)),
            pl.BlockSpec((_BI * _H, _F), lambda g: (0, 0)),
            pl.BlockSpec((_BI * _H, 1), lambda g: (0, 0)),
            pl.BlockSpec((_BI * _H, 1), lambda g: (0, 0)),
            pl.BlockSpec((1, 1), lambda g: (0, 0)),
        ],
        out_specs=[
            pl.BlockSpec((_BI, _N), lambda g: (g, 0)),
            pl.BlockSpec((_BI, _N), lambda g: (g, 0)),
        ],
        out_shape=[
            jax.ShapeDtypeStruct((_N, _N), f32),
            jax.ShapeDtypeStruct((_N, _N), f32),
        ],
    )(h, hT, W5Tt, b5c, w6c, b6d)
    return jnp.stack([out0, out1], axis=-1)


# speedup vs baseline: 1.0017x; 1.0017x over previous
import jax
import jax.numpy as jnp
from jax.experimental import pallas as pl
from jax.experimental.pallas import tpu as pltpu

_N = 1024
_F = 64
_H = 128
_BI = 8


def _fused_kernel(nf, W1, b1, W2, b2, W4, b4, W5Tt, b5c, w6c, b6d,
                  out0, out1, h_s, hT_s):
    g = pl.program_id(0)

    @pl.when(g == 0)
    def _node_stage():
        h0 = nf[...]
        t = jnp.dot(h0, W1[...], preferred_element_type=jnp.float32) + b1[...]
        t = jnp.where(t >= 0.0, t, 0.01 * t)
        t = jnp.dot(t, W2[...], preferred_element_type=jnp.float32) + b2[...]
        t = jnp.where(t >= 0.0, t, 0.01 * t)
        t = jnp.dot(t, W4[...], preferred_element_type=jnp.float32) + b4[...]
        h = t + h0
        h_s[...] = h
        hT_s[...] = h.T.astype(jnp.bfloat16)

    hi = h_s[pl.ds(g * _BI, _BI), :]
    hrep = jax.lax.broadcast_in_dim(hi, (_BI, _H, _F), (0, 2))
    hrep = hrep.reshape(_BI * _H, _F)
    A = (hrep * W5Tt[...]).astype(jnp.bfloat16)
    T = jnp.dot(A, hT_s[...], preferred_element_type=jnp.float32)
    T = jnp.maximum(T + b5c[...], 0.0)
    U = T * w6c[...]
    D = jnp.sum(U.reshape(_BI, _H, _N), axis=1) + b6d[0, 0]
    p1 = jax.nn.sigmoid(D)
    p0 = 1.0 - p1
    row = jax.lax.broadcasted_iota(jnp.int32, (_BI, _N), 0)
    col = jax.lax.broadcasted_iota(jnp.int32, (_BI, _N), 1)
    diag = col == (g * _BI + row)
    out0[...] = jnp.where(diag, 0.0, p0)
    out1[...] = jnp.where(diag, 0.0, p1)


def kernel(x, node_features, W1, b1, W2, b2, W4, b4, W5, b5, W6, b6):
    f32 = jnp.float32
    W5Tt = jnp.tile(W5.T, (_BI, 1))
    b5c = jnp.tile(b5, _BI).reshape(_BI * _H, 1)
    w6c = jnp.tile(W6[:, 1] - W6[:, 0], _BI).reshape(_BI * _H, 1)
    b6d = (b6[1] - b6[0]).reshape(1, 1)

    full = lambda shape: pl.BlockSpec(shape, lambda g: tuple(0 for _ in shape))
    out = pl.pallas_call(
        _fused_kernel,
        grid=(_N // _BI,),
        in_specs=[
            full((_N, _F)),
            full((_F, _H)),
            full((1, _H)),
            full((_H, _H)),
            full((1, _H)),
            full((_H, _F)),
            full((1, _F)),
            full((_BI * _H, _F)),
            full((_BI * _H, 1)),
            full((_BI * _H, 1)),
            full((1, 1)),
        ],
        out_specs=[
            pl.BlockSpec((_BI, _N), lambda g: (g, 0)),
            pl.BlockSpec((_BI, _N), lambda g: (g, 0)),
        ],
        out_shape=[
            jax.ShapeDtypeStruct((_N, _N), f32),
            jax.ShapeDtypeStruct((_N, _N), f32),
        ],
        scratch_shapes=[
            pltpu.VMEM((_N, _F), f32),
            pltpu.VMEM((_F, _N), jnp.bfloat16),
        ],
    )(
        node_features, W1, b1.reshape(1, _H), W2, b2.reshape(1, _H),
        W4, b4.reshape(1, _F), W5Tt, b5c, w6c, b6d,
    )
    out0, out1 = out
    return jnp.stack([out0, out1], axis=-1)


# speedup vs baseline: 1.1975x; 1.1954x over previous
import jax
import jax.numpy as jnp
from jax.experimental import pallas as pl
from jax.experimental.pallas import tpu as pltpu

_N = 1024
_F = 64
_H = 128
_BI = 8

_PHASES = (
    (0, 32, 0, 1024),
    (32, 64, 256, 768),
    (64, 96, 512, 512),
    (96, 128, 768, 256),
)
_TRANSPOSES = (
    (256, 0, 256),
    (512, 0, 512),
    (768, 0, 768),
)


def _fused_kernel(nf, W1, b1, W2, b2, W4, b4, W5Tt, b5c, w6c, b6d,
                  out0, out1, h_s, hT_s):
    t = pl.program_id(0)

    @pl.when(t == 0)
    def _node_stage():
        h0 = nf[...]
        z = jnp.dot(h0, W1[...], preferred_element_type=jnp.float32) + b1[...]
        z = jnp.where(z >= 0.0, z, 0.01 * z)
        z = jnp.dot(z, W2[...], preferred_element_type=jnp.float32) + b2[...]
        z = jnp.where(z >= 0.0, z, 0.01 * z)
        z = jnp.dot(z, W4[...], preferred_element_type=jnp.float32) + b4[...]
        h = z + h0
        h_s[...] = h
        hT_s[...] = h.T.astype(jnp.bfloat16)

    for (lo, hi, c0, w) in _PHASES:
        @pl.when((t >= lo) & (t < hi))
        def _compute(c0=c0, w=w):
            hi_rows = h_s[pl.ds(t * _BI, _BI), :]
            hrep = jax.lax.broadcast_in_dim(hi_rows, (_BI, _H, _F), (0, 2))
            hrep = hrep.reshape(_BI * _H, _F)
            A = (hrep * W5Tt[...]).astype(jnp.bfloat16)
            T = jnp.dot(A, hT_s[:, c0:c0 + w],
                        preferred_element_type=jnp.float32)
            T = jnp.maximum(T + b5c[...], 0.0)
            U = T * w6c[...]
            D = jnp.sum(U.reshape(_BI, _H, w), axis=1) + b6d[0, 0]
            p1 = jax.nn.sigmoid(D)
            p0 = 1.0 - p1
            row = jax.lax.broadcasted_iota(jnp.int32, (_BI, w), 0)
            col = jax.lax.broadcasted_iota(jnp.int32, (_BI, w), 1)
            diag = (col + c0) == (t * _BI + row)
            r0 = t * _BI
            out0[pl.ds(r0, _BI), c0:c0 + w] = jnp.where(diag, 0.0, p0)
            out1[pl.ds(r0, _BI), c0:c0 + w] = jnp.where(diag, 0.0, p1)

    @pl.when(t == 128)
    def _mirror():
        for (dr, dc, w) in _TRANSPOSES:
            out0[dr:dr + 256, dc:dc + w] = out0[dc:dc + w, dr:dr + 256].T
            out1[dr:dr + 256, dc:dc + w] = out1[dc:dc + w, dr:dr + 256].T


def kernel(x, node_features, W1, b1, W2, b2, W4, b4, W5, b5, W6, b6):
    f32 = jnp.float32
    W5Tt = jnp.tile(W5.T, (_BI, 1))
    b5c = jnp.tile(b5, _BI).reshape(_BI * _H, 1)
    w6c = jnp.tile(W6[:, 1] - W6[:, 0], _BI).reshape(_BI * _H, 1)
    b6d = (b6[1] - b6[0]).reshape(1, 1)

    full = lambda shape: pl.BlockSpec(shape, lambda g: tuple(0 for _ in shape))
    out0, out1 = pl.pallas_call(
        _fused_kernel,
        grid=(129,),
        in_specs=[
            full((_N, _F)),
            full((_F, _H)),
            full((1, _H)),
            full((_H, _H)),
            full((1, _H)),
            full((_H, _F)),
            full((1, _F)),
            full((_BI * _H, _F)),
            full((_BI * _H, 1)),
            full((_BI * _H, 1)),
            full((1, 1)),
        ],
        out_specs=[
            pl.BlockSpec((_N, _N), lambda g: (0, 0)),
            pl.BlockSpec((_N, _N), lambda g: (0, 0)),
        ],
        out_shape=[
            jax.ShapeDtypeStruct((_N, _N), f32),
            jax.ShapeDtypeStruct((_N, _N), f32),
        ],
        scratch_shapes=[
            pltpu.VMEM((_N, _F), f32),
            pltpu.VMEM((_F, _N), jnp.bfloat16),
        ],
    )(
        node_features, W1, b1.reshape(1, _H), W2, b2.reshape(1, _H),
        W4, b4.reshape(1, _F), W5Tt, b5c, w6c, b6d,
    )
    return jnp.stack([out0, out1], axis=-1)


# speedup vs baseline: 1.3085x; 1.0927x over previous
import jax
import jax.numpy as jnp
from jax.experimental import pallas as pl
from jax.experimental.pallas import tpu as pltpu

_N = 1024
_F = 64
_H = 128
_BI = 16
_Q = 256 // _BI

_PHASES = (
    (0 * _Q, 1 * _Q, 0, 1024),
    (1 * _Q, 2 * _Q, 256, 768),
    (2 * _Q, 3 * _Q, 512, 512),
    (3 * _Q, 4 * _Q, 768, 256),
)
_TRANSPOSES = (
    (256, 0, 256),
    (512, 0, 512),
    (768, 0, 768),
)


def _fused_kernel(nf, W1, b1, W2, b2, W4, b4, W5Tt, b5c, w6c, b6d,
                  out0, out1, h_s, hT_s):
    t = pl.program_id(0)

    @pl.when(t == 0)
    def _node_stage():
        h0 = nf[...]
        z = jnp.dot(h0, W1[...], preferred_element_type=jnp.float32) + b1[...]
        z = jnp.where(z >= 0.0, z, 0.01 * z)
        z = jnp.dot(z, W2[...], preferred_element_type=jnp.float32) + b2[...]
        z = jnp.where(z >= 0.0, z, 0.01 * z)
        z = jnp.dot(z, W4[...], preferred_element_type=jnp.float32) + b4[...]
        h = z + h0
        h_s[...] = h
        hT_s[...] = h.T.astype(jnp.bfloat16)

    for (lo, hi, c0, w) in _PHASES:
        @pl.when((t >= lo) & (t < hi))
        def _compute(c0=c0, w=w):
            hi_rows = h_s[pl.ds(t * _BI, _BI), :]
            hrep = jax.lax.broadcast_in_dim(hi_rows, (_BI, _H, _F), (0, 2))
            hrep = hrep.reshape(_BI * _H, _F)
            A = (hrep * W5Tt[...]).astype(jnp.bfloat16)
            T = jnp.dot(A, hT_s[:, c0:c0 + w],
                        preferred_element_type=jnp.float32)
            T = jnp.maximum(T + b5c[...], 0.0)
            U = T * w6c[...]
            D = jnp.sum(U.reshape(_BI, _H, w), axis=1) + b6d[0, 0]
            p1 = jax.nn.sigmoid(D)
            p0 = 1.0 - p1
            row = jax.lax.broadcasted_iota(jnp.int32, (_BI, w), 0)
            col = jax.lax.broadcasted_iota(jnp.int32, (_BI, w), 1)
            diag = (col + c0) == (t * _BI + row)
            r0 = t * _BI
            out0[pl.ds(r0, _BI), c0:c0 + w] = jnp.where(diag, 0.0, p0)
            out1[pl.ds(r0, _BI), c0:c0 + w] = jnp.where(diag, 0.0, p1)

    @pl.when(t == 4 * _Q)
    def _mirror():
        for (dr, dc, w) in _TRANSPOSES:
            out0[dr:dr + 256, dc:dc + w] = out0[dc:dc + w, dr:dr + 256].T
            out1[dr:dr + 256, dc:dc + w] = out1[dc:dc + w, dr:dr + 256].T


def kernel(x, node_features, W1, b1, W2, b2, W4, b4, W5, b5, W6, b6):
    f32 = jnp.float32
    W5Tt = jnp.tile(W5.T, (_BI, 1))
    b5c = jnp.tile(b5, _BI).reshape(_BI * _H, 1)
    w6c = jnp.tile(W6[:, 1] - W6[:, 0], _BI).reshape(_BI * _H, 1)
    b6d = (b6[1] - b6[0]).reshape(1, 1)

    full = lambda shape: pl.BlockSpec(shape, lambda g: tuple(0 for _ in shape))
    out0, out1 = pl.pallas_call(
        _fused_kernel,
        grid=(4 * _Q + 1,),
        in_specs=[
            full((_N, _F)),
            full((_F, _H)),
            full((1, _H)),
            full((_H, _H)),
            full((1, _H)),
            full((_H, _F)),
            full((1, _F)),
            full((_BI * _H, _F)),
            full((_BI * _H, 1)),
            full((_BI * _H, 1)),
            full((1, 1)),
        ],
        out_specs=[
            pl.BlockSpec((_N, _N), lambda g: (0, 0)),
            pl.BlockSpec((_N, _N), lambda g: (0, 0)),
        ],
        out_shape=[
            jax.ShapeDtypeStruct((_N, _N), f32),
            jax.ShapeDtypeStruct((_N, _N), f32),
        ],
        scratch_shapes=[
            pltpu.VMEM((_N, _F), f32),
            pltpu.VMEM((_F, _N), jnp.bfloat16),
        ],
    )(
        node_features, W1, b1.reshape(1, _H), W2, b2.reshape(1, _H),
        W4, b4.reshape(1, _F), W5Tt, b5c, w6c, b6d,
    )
    return jnp.stack([out0, out1], axis=-1)


# speedup vs baseline: 1.3304x; 1.0167x over previous
import jax
import jax.numpy as jnp
from jax.experimental import pallas as pl
from jax.experimental.pallas import tpu as pltpu

_N = 1024
_F = 64
_H = 128
_BI = 32
_Q = 256 // _BI

_PHASES = (
    (0 * _Q, 1 * _Q, 0, 1024),
    (1 * _Q, 2 * _Q, 256, 768),
    (2 * _Q, 3 * _Q, 512, 512),
    (3 * _Q, 4 * _Q, 768, 256),
)
_TRANSPOSES = (
    (256, 0, 256),
    (512, 0, 512),
    (768, 0, 768),
)


def _fused_kernel(nf, W1, b1, W2, b2, W4, b4, W5Tt, b5c, w6c, b6d,
                  out0, out1, h_s, hT_s):
    t = pl.program_id(0)

    @pl.when(t == 0)
    def _node_stage():
        h0 = nf[...]
        z = jnp.dot(h0, W1[...], preferred_element_type=jnp.float32) + b1[...]
        z = jnp.where(z >= 0.0, z, 0.01 * z)
        z = jnp.dot(z, W2[...], preferred_element_type=jnp.float32) + b2[...]
        z = jnp.where(z >= 0.0, z, 0.01 * z)
        z = jnp.dot(z, W4[...], preferred_element_type=jnp.float32) + b4[...]
        h = z + h0
        h_s[...] = h
        hT_s[...] = h.T.astype(jnp.bfloat16)

    for (lo, hi, c0, w) in _PHASES:
        @pl.when((t >= lo) & (t < hi))
        def _compute(c0=c0, w=w):
            hi_rows = h_s[pl.ds(t * _BI, _BI), :]
            hrep = jax.lax.broadcast_in_dim(hi_rows, (_BI, _H, _F), (0, 2))
            hrep = hrep.reshape(_BI * _H, _F)
            A = (hrep * W5Tt[...]).astype(jnp.bfloat16)
            T = jnp.dot(A, hT_s[:, c0:c0 + w],
                        preferred_element_type=jnp.float32)
            T = jnp.maximum(T + b5c[...], 0.0)
            U = T * w6c[...]
            D = jnp.sum(U.reshape(_BI, _H, w), axis=1) + b6d[0, 0]
            p1 = jax.nn.sigmoid(D)
            p0 = 1.0 - p1
            row = jax.lax.broadcasted_iota(jnp.int32, (_BI, w), 0)
            col = jax.lax.broadcasted_iota(jnp.int32, (_BI, w), 1)
            diag = (col + c0) == (t * _BI + row)
            r0 = t * _BI
            out0[pl.ds(r0, _BI), c0:c0 + w] = jnp.where(diag, 0.0, p0)
            out1[pl.ds(r0, _BI), c0:c0 + w] = jnp.where(diag, 0.0, p1)

    @pl.when(t == 4 * _Q)
    def _mirror():
        for (dr, dc, w) in _TRANSPOSES:
            out0[dr:dr + 256, dc:dc + w] = out0[dc:dc + w, dr:dr + 256].T
            out1[dr:dr + 256, dc:dc + w] = out1[dc:dc + w, dr:dr + 256].T


def kernel(x, node_features, W1, b1, W2, b2, W4, b4, W5, b5, W6, b6):
    f32 = jnp.float32
    W5Tt = jnp.tile(W5.T, (_BI, 1))
    b5c = jnp.tile(b5, _BI).reshape(_BI * _H, 1)
    w6c = jnp.tile(W6[:, 1] - W6[:, 0], _BI).reshape(_BI * _H, 1)
    b6d = (b6[1] - b6[0]).reshape(1, 1)

    full = lambda shape: pl.BlockSpec(shape, lambda g: tuple(0 for _ in shape))
    out0, out1 = pl.pallas_call(
        _fused_kernel,
        grid=(4 * _Q + 1,),
        in_specs=[
            full((_N, _F)),
            full((_F, _H)),
            full((1, _H)),
            full((_H, _H)),
            full((1, _H)),
            full((_H, _F)),
            full((1, _F)),
            full((_BI * _H, _F)),
            full((_BI * _H, 1)),
            full((_BI * _H, 1)),
            full((1, 1)),
        ],
        out_specs=[
            pl.BlockSpec((_N, _N), lambda g: (0, 0)),
            pl.BlockSpec((_N, _N), lambda g: (0, 0)),
        ],
        out_shape=[
            jax.ShapeDtypeStruct((_N, _N), f32),
            jax.ShapeDtypeStruct((_N, _N), f32),
        ],
        scratch_shapes=[
            pltpu.VMEM((_N, _F), f32),
            pltpu.VMEM((_F, _N), jnp.bfloat16),
        ],
    )(
        node_features, W1, b1.reshape(1, _H), W2, b2.reshape(1, _H),
        W4, b4.reshape(1, _F), W5Tt, b5c, w6c, b6d,
    )
    return jnp.stack([out0, out1], axis=-1)
